# baseline (speedup 1.0000x reference)
import functools

import jax
import jax.numpy as jnp
from jax.experimental import pallas as pl
from jax.experimental.pallas import tpu as pltpu

N = 32768
IN_FEAT = 768
OUT_FEAT = 768
CHUNK = 2048
NBUF = 4
NCHUNK = N // CHUNK


def _body(m_ref, w_ref, b_ref, x_hbm, o_hbm, xbuf, ybuf, wb, rsem, wsem):
    i = pl.program_id(0)
    slot = jax.lax.rem(i, NBUF)

    @pl.when(i == 0)
    def _prime():
        wb[...] = w_ref[...].astype(jnp.bfloat16)
        for j in range(NBUF):
            pltpu.make_async_copy(
                x_hbm.at[pl.ds(j * CHUNK, CHUNK)], xbuf.at[j], rsem.at[j]
            ).start()

    pltpu.make_async_copy(
        x_hbm.at[pl.ds(i * CHUNK, CHUNK)], xbuf.at[slot], rsem.at[slot]
    ).wait()

    @pl.when(i >= NBUF)
    def _drain_prev_write():
        pltpu.make_async_copy(
            ybuf.at[slot], o_hbm.at[pl.ds((i - NBUF) * CHUNK, CHUNK)], wsem.at[slot]
        ).wait()

    xb = xbuf[slot].astype(jnp.bfloat16)
    y = jax.lax.dot_general(
        xb, wb[...], (((1,), (1,)), ((), ())),
        preferred_element_type=jnp.float32,
    )
    y = y + b_ref[...]
    ybuf[slot] = jnp.where(m_ref[...] != 0, y, 0.0)

    pltpu.make_async_copy(
        ybuf.at[slot], o_hbm.at[pl.ds(i * CHUNK, CHUNK)], wsem.at[slot]
    ).start()

    @pl.when(i + NBUF < NCHUNK)
    def _next_read():
        pltpu.make_async_copy(
            x_hbm.at[pl.ds((i + NBUF) * CHUNK, CHUNK)], xbuf.at[slot], rsem.at[slot]
        ).start()

    @pl.when(i == NCHUNK - 1)
    def _drain_tail():
        for j in range(NBUF):
            c = NCHUNK - NBUF + j
            s = c % NBUF
            pltpu.make_async_copy(
                ybuf.at[s], o_hbm.at[pl.ds(c * CHUNK, CHUNK)], wsem.at[s]
            ).wait()


@jax.jit
def _masked_linear(x, amask2d, w, b2d):
    return pl.pallas_call(
        _body,
        grid=(NCHUNK,),
        in_specs=[
            pl.BlockSpec((CHUNK, 1), lambda i: (i, 0)),
            pl.BlockSpec((OUT_FEAT, IN_FEAT), lambda i: (0, 0)),
            pl.BlockSpec((1, OUT_FEAT), lambda i: (0, 0)),
            pl.BlockSpec(memory_space=pl.ANY),
        ],
        out_specs=pl.BlockSpec(memory_space=pl.ANY),
        out_shape=jax.ShapeDtypeStruct((N, OUT_FEAT), jnp.float32),
        scratch_shapes=[
            pltpu.VMEM((NBUF, CHUNK, IN_FEAT), jnp.float32),
            pltpu.VMEM((NBUF, CHUNK, OUT_FEAT), jnp.float32),
            pltpu.VMEM((OUT_FEAT, IN_FEAT), jnp.bfloat16),
            pltpu.SemaphoreType.DMA((NBUF,)),
            pltpu.SemaphoreType.DMA((NBUF,)),
        ],
        compiler_params=pltpu.CompilerParams(
            dimension_semantics=("arbitrary",),
        ),
    )(amask2d, w, b2d, x)


def kernel(x, amask, W, b):
    return _masked_linear(
        x,
        amask.reshape(N, 1),
        W,
        b.reshape(1, OUT_FEAT),
    )


# speedup vs baseline: 1.2277x; 1.2277x over previous
import jax
import jax.numpy as jnp
from jax.experimental import pallas as pl
from jax.experimental.pallas import tpu as pltpu

N = 32768
IN_FEAT = 768
OUT_FEAT = 768
CHUNK = 2048
NBUF = 4
NCHUNK = N // CHUNK


def _body(m_hbm, w_hbm, b_hbm, x_hbm, o_hbm,
          xbuf, ybuf, mbuf, wf32, wb, bbuf, rsem, wsem, csem):
    i = pl.program_id(0)
    slot = jax.lax.rem(i, NBUF)

    @pl.when(i == 0)
    def _prime():
        for j in range(NBUF):
            pltpu.make_async_copy(
                x_hbm.at[pl.ds(j * CHUNK, CHUNK)], xbuf.at[j], rsem.at[j]
            ).start()
        pltpu.make_async_copy(m_hbm, mbuf, csem.at[0]).start()
        pltpu.make_async_copy(w_hbm, wf32, csem.at[1]).start()
        pltpu.make_async_copy(b_hbm, bbuf, csem.at[2]).start()
        pltpu.make_async_copy(m_hbm, mbuf, csem.at[0]).wait()
        pltpu.make_async_copy(w_hbm, wf32, csem.at[1]).wait()
        pltpu.make_async_copy(b_hbm, bbuf, csem.at[2]).wait()
        wb[...] = wf32[...].astype(jnp.bfloat16)

    pltpu.make_async_copy(
        x_hbm.at[pl.ds(i * CHUNK, CHUNK)], xbuf.at[slot], rsem.at[slot]
    ).wait()

    @pl.when(i >= NBUF)
    def _drain_prev_write():
        pltpu.make_async_copy(
            ybuf.at[slot], o_hbm.at[pl.ds((i - NBUF) * CHUNK, CHUNK)], wsem.at[slot]
        ).wait()

    xb = xbuf[slot].astype(jnp.bfloat16)
    y = jax.lax.dot_general(
        xb, wb[...], (((1,), (1,)), ((), ())),
        preferred_element_type=jnp.float32,
    )
    y = y + bbuf[...]
    mcol = mbuf[i, :].reshape(CHUNK, 1)
    ybuf[slot] = jnp.where(mcol != 0, y, 0.0)

    pltpu.make_async_copy(
        ybuf.at[slot], o_hbm.at[pl.ds(i * CHUNK, CHUNK)], wsem.at[slot]
    ).start()

    @pl.when(i + NBUF < NCHUNK)
    def _next_read():
        pltpu.make_async_copy(
            x_hbm.at[pl.ds((i + NBUF) * CHUNK, CHUNK)], xbuf.at[slot], rsem.at[slot]
        ).start()

    @pl.when(i == NCHUNK - 1)
    def _drain_tail():
        for j in range(NBUF):
            c = NCHUNK - NBUF + j
            s = c % NBUF
            pltpu.make_async_copy(
                ybuf.at[s], o_hbm.at[pl.ds(c * CHUNK, CHUNK)], wsem.at[s]
            ).wait()


@jax.jit
def _masked_linear(x, amask2d, w, b2d):
    return pl.pallas_call(
        _body,
        grid=(NCHUNK,),
        in_specs=[
            pl.BlockSpec(memory_space=pl.ANY),
            pl.BlockSpec(memory_space=pl.ANY),
            pl.BlockSpec(memory_space=pl.ANY),
            pl.BlockSpec(memory_space=pl.ANY),
        ],
        out_specs=pl.BlockSpec(memory_space=pl.ANY),
        out_shape=jax.ShapeDtypeStruct((N, OUT_FEAT), jnp.float32),
        scratch_shapes=[
            pltpu.VMEM((NBUF, CHUNK, IN_FEAT), jnp.float32),
            pltpu.VMEM((NBUF, CHUNK, OUT_FEAT), jnp.float32),
            pltpu.VMEM((NCHUNK, CHUNK), jnp.int32),
            pltpu.VMEM((OUT_FEAT, IN_FEAT), jnp.float32),
            pltpu.VMEM((OUT_FEAT, IN_FEAT), jnp.bfloat16),
            pltpu.VMEM((1, OUT_FEAT), jnp.float32),
            pltpu.SemaphoreType.DMA((NBUF,)),
            pltpu.SemaphoreType.DMA((NBUF,)),
            pltpu.SemaphoreType.DMA((3,)),
        ],
        compiler_params=pltpu.CompilerParams(
            dimension_semantics=("arbitrary",),
        ),
    )(amask2d, w, b2d, x)


def kernel(x, amask, W, b):
    return _masked_linear(
        x,
        amask.reshape(NCHUNK, CHUNK),
        W,
        b.reshape(1, OUT_FEAT),
    )


# speedup vs baseline: 1.2293x; 1.0013x over previous
import jax
import jax.numpy as jnp
from jax.experimental import pallas as pl
from jax.experimental.pallas import tpu as pltpu

N = 32768
IN_FEAT = 768
OUT_FEAT = 768
CHUNK = 1024
NBUF = 6
NCHUNK = N // CHUNK


def _body(m_hbm, w_hbm, b_hbm, x_hbm, o_hbm,
          xbuf, ybuf, mbuf, wf32, wb, bbuf, rsem, wsem, csem):
    i = pl.program_id(0)
    slot = jax.lax.rem(i, NBUF)

    @pl.when(i == 0)
    def _prime():
        for j in range(NBUF):
            pltpu.make_async_copy(
                x_hbm.at[pl.ds(j * CHUNK, CHUNK)], xbuf.at[j], rsem.at[j]
            ).start()
        pltpu.make_async_copy(m_hbm, mbuf, csem.at[0]).start()
        pltpu.make_async_copy(w_hbm, wf32, csem.at[1]).start()
        pltpu.make_async_copy(b_hbm, bbuf, csem.at[2]).start()
        pltpu.make_async_copy(m_hbm, mbuf, csem.at[0]).wait()
        pltpu.make_async_copy(w_hbm, wf32, csem.at[1]).wait()
        pltpu.make_async_copy(b_hbm, bbuf, csem.at[2]).wait()
        wb[...] = wf32[...].astype(jnp.bfloat16)

    pltpu.make_async_copy(
        x_hbm.at[pl.ds(i * CHUNK, CHUNK)], xbuf.at[slot], rsem.at[slot]
    ).wait()

    @pl.when(i >= NBUF)
    def _drain_prev_write():
        pltpu.make_async_copy(
            ybuf.at[slot], o_hbm.at[pl.ds((i - NBUF) * CHUNK, CHUNK)], wsem.at[slot]
        ).wait()

    xb = xbuf[slot].astype(jnp.bfloat16)
    y = jax.lax.dot_general(
        xb, wb[...], (((1,), (1,)), ((), ())),
        preferred_element_type=jnp.float32,
    )
    y = y + bbuf[...]
    mcol = mbuf[i, :].reshape(CHUNK, 1)
    ybuf[slot] = jnp.where(mcol != 0, y, 0.0)

    pltpu.make_async_copy(
        ybuf.at[slot], o_hbm.at[pl.ds(i * CHUNK, CHUNK)], wsem.at[slot]
    ).start()

    @pl.when(i + NBUF < NCHUNK)
    def _next_read():
        pltpu.make_async_copy(
            x_hbm.at[pl.ds((i + NBUF) * CHUNK, CHUNK)], xbuf.at[slot], rsem.at[slot]
        ).start()

    @pl.when(i == NCHUNK - 1)
    def _drain_tail():
        for j in range(NBUF):
            c = NCHUNK - NBUF + j
            s = c % NBUF
            pltpu.make_async_copy(
                ybuf.at[s], o_hbm.at[pl.ds(c * CHUNK, CHUNK)], wsem.at[s]
            ).wait()


@jax.jit
def _masked_linear(x, amask2d, w, b2d):
    return pl.pallas_call(
        _body,
        grid=(NCHUNK,),
        in_specs=[
            pl.BlockSpec(memory_space=pl.ANY),
            pl.BlockSpec(memory_space=pl.ANY),
            pl.BlockSpec(memory_space=pl.ANY),
            pl.BlockSpec(memory_space=pl.ANY),
        ],
        out_specs=pl.BlockSpec(memory_space=pl.ANY),
        out_shape=jax.ShapeDtypeStruct((N, OUT_FEAT), jnp.float32),
        scratch_shapes=[
            pltpu.VMEM((NBUF, CHUNK, IN_FEAT), jnp.float32),
            pltpu.VMEM((NBUF, CHUNK, OUT_FEAT), jnp.float32),
            pltpu.VMEM((NCHUNK, CHUNK), jnp.int32),
            pltpu.VMEM((OUT_FEAT, IN_FEAT), jnp.float32),
            pltpu.VMEM((OUT_FEAT, IN_FEAT), jnp.bfloat16),
            pltpu.VMEM((1, OUT_FEAT), jnp.float32),
            pltpu.SemaphoreType.DMA((NBUF,)),
            pltpu.SemaphoreType.DMA((NBUF,)),
            pltpu.SemaphoreType.DMA((3,)),
        ],
        compiler_params=pltpu.CompilerParams(
            dimension_semantics=("arbitrary",),
        ),
    )(amask2d, w, b2d, x)


def kernel(x, amask, W, b):
    return _masked_linear(
        x,
        amask.reshape(NCHUNK, CHUNK),
        W,
        b.reshape(1, OUT_FEAT),
    )


# speedup vs baseline: 1.2366x; 1.0059x over previous
import jax
import jax.numpy as jnp
from jax.experimental import pallas as pl
from jax.experimental.pallas import tpu as pltpu

N = 32768
IN_FEAT = 768
OUT_FEAT = 768
CHUNK = 1024
NBUF = 8
NCHUNK = N // CHUNK


def _body(m_hbm, w_hbm, b_hbm, x_hbm, o_hbm,
          xbuf, ybuf, mbuf, wf32, wb, bbuf, rsem, wsem, csem):
    i = pl.program_id(0)
    slot = jax.lax.rem(i, NBUF)

    @pl.when(i == 0)
    def _prime():
        for j in range(NBUF):
            pltpu.make_async_copy(
                x_hbm.at[pl.ds(j * CHUNK, CHUNK)], xbuf.at[j], rsem.at[j]
            ).start()
        pltpu.make_async_copy(m_hbm, mbuf, csem.at[0]).start()
        pltpu.make_async_copy(w_hbm, wf32, csem.at[1]).start()
        pltpu.make_async_copy(b_hbm, bbuf, csem.at[2]).start()
        pltpu.make_async_copy(m_hbm, mbuf, csem.at[0]).wait()
        pltpu.make_async_copy(w_hbm, wf32, csem.at[1]).wait()
        pltpu.make_async_copy(b_hbm, bbuf, csem.at[2]).wait()
        wb[...] = wf32[...].astype(jnp.bfloat16)

    pltpu.make_async_copy(
        x_hbm.at[pl.ds(i * CHUNK, CHUNK)], xbuf.at[slot], rsem.at[slot]
    ).wait()

    @pl.when(i >= NBUF)
    def _drain_prev_write():
        pltpu.make_async_copy(
            ybuf.at[slot], o_hbm.at[pl.ds((i - NBUF) * CHUNK, CHUNK)], wsem.at[slot]
        ).wait()

    xb = xbuf[slot].astype(jnp.bfloat16)
    y = jax.lax.dot_general(
        xb, wb[...], (((1,), (1,)), ((), ())),
        preferred_element_type=jnp.float32,
    )
    y = y + bbuf[...]
    mcol = mbuf[i, :].reshape(CHUNK, 1)
    ybuf[slot] = jnp.where(mcol != 0, y, 0.0)

    pltpu.make_async_copy(
        ybuf.at[slot], o_hbm.at[pl.ds(i * CHUNK, CHUNK)], wsem.at[slot]
    ).start()

    @pl.when(i + NBUF < NCHUNK)
    def _next_read():
        pltpu.make_async_copy(
            x_hbm.at[pl.ds((i + NBUF) * CHUNK, CHUNK)], xbuf.at[slot], rsem.at[slot]
        ).start()

    @pl.when(i == NCHUNK - 1)
    def _drain_tail():
        for j in range(NBUF):
            c = NCHUNK - NBUF + j
            s = c % NBUF
            pltpu.make_async_copy(
                ybuf.at[s], o_hbm.at[pl.ds(c * CHUNK, CHUNK)], wsem.at[s]
            ).wait()


@jax.jit
def _masked_linear(x, amask2d, w, b2d):
    return pl.pallas_call(
        _body,
        grid=(NCHUNK,),
        in_specs=[
            pl.BlockSpec(memory_space=pl.ANY),
            pl.BlockSpec(memory_space=pl.ANY),
            pl.BlockSpec(memory_space=pl.ANY),
            pl.BlockSpec(memory_space=pl.ANY),
        ],
        out_specs=pl.BlockSpec(memory_space=pl.ANY),
        out_shape=jax.ShapeDtypeStruct((N, OUT_FEAT), jnp.float32),
        scratch_shapes=[
            pltpu.VMEM((NBUF, CHUNK, IN_FEAT), jnp.float32),
            pltpu.VMEM((NBUF, CHUNK, OUT_FEAT), jnp.float32),
            pltpu.VMEM((NCHUNK, CHUNK), jnp.int32),
            pltpu.VMEM((OUT_FEAT, IN_FEAT), jnp.float32),
            pltpu.VMEM((OUT_FEAT, IN_FEAT), jnp.bfloat16),
            pltpu.VMEM((1, OUT_FEAT), jnp.float32),
            pltpu.SemaphoreType.DMA((NBUF,)),
            pltpu.SemaphoreType.DMA((NBUF,)),
            pltpu.SemaphoreType.DMA((3,)),
        ],
        compiler_params=pltpu.CompilerParams(
            dimension_semantics=("arbitrary",),
        ),
    )(amask2d, w, b2d, x)


def kernel(x, amask, W, b):
    return _masked_linear(
        x,
        amask.reshape(NCHUNK, CHUNK),
        W,
        b.reshape(1, OUT_FEAT),
    )
